# baseline (speedup 1.0000x reference)
import functools

import jax
import jax.numpy as jnp
from jax import lax
from jax.experimental import pallas as pl
from jax.experimental.pallas import tpu as pltpu
from jax.experimental.pallas import tpu_sc as plsc

N = 10000
E = 320000
D = 128
NP = 10000
DROWS = 1264
NPD = NP + DROWS
NC, NS, L = 2, 16, 16
NW = NC * NS
EPT = E // NW
B = 48
NCH_MAIN = EPT // B
TAIL = EPT - NCH_MAIN * B
NPAIR = NCH_MAIN // 2
RPS = NPD // NS


def _make_edge_kernel(heads):
    mesh = plsc.VectorSubcoreMesh(core_axis_name="c", subcore_axis_name="s",
                                  num_cores=NC, num_subcores=NS)

    @functools.partial(
        pl.kernel,
        mesh=mesh,
        out_type=jax.ShapeDtypeStruct((NC, NPD, D), jnp.float32),
        scratch_types=[
            pltpu.VMEM((B,), jnp.int32),
            pltpu.VMEM((B,), jnp.int32),
            pltpu.VMEM((B,), jnp.int32),
            pltpu.VMEM((B,), jnp.int32),
            pltpu.VMEM((B + L,), jnp.int32),
            pltpu.VMEM((B,), jnp.int32),
            pltpu.VMEM((B,), jnp.int32),
            pltpu.VMEM((B, D), jnp.float32),
            pltpu.VMEM((B, D), jnp.float32),
            pltpu.VMEM((B, D), jnp.float32),
            pltpu.VMEM((B, D), jnp.float32),
            pltpu.VMEM((B, D), jnp.float32),
            pltpu.VMEM((B, D), jnp.float32),
            pltpu.VMEM((2 * D,), jnp.float32),
            pltpu.VMEM_SHARED((NPD, D), jnp.float32),
            pltpu.SemaphoreType.DMA,
            pltpu.SemaphoreType.DMA,
            pltpu.SemaphoreType.DMA,
            pltpu.SemaphoreType.DMA,
            pltpu.SemaphoreType.DMA,
            pltpu.SemaphoreType.DMA,
        ],
    )
    def k(xl_hbm, xr_hbm, src_hbm, dst_hbm, att_hbm, z1_hbm, out_hbm,
          sidxA, sidxB, didxA, didxB, didxp, didx2A, didx2B,
          glA, glB, grA, grB, cbA, cbB, attv, acc,
          semA1, semA2, semB1, semB2, semSA, semSB):
        c = lax.axis_index("c")
        s = lax.axis_index("s")
        wid = s * NC + c
        pltpu.sync_copy(z1_hbm.at[pl.ds(s * RPS, RPS)],
                        acc.at[pl.ds(s * RPS, RPS)])
        pltpu.sync_copy(att_hbm, attv)
        plsc.subcore_barrier()

        ebase = wid * EPT
        lanes = lax.iota(jnp.int32, L)
        zero16 = jnp.zeros((L,), jnp.float32)

        def onehot(h):
            return attv[pl.ds(D + h * 16, 16)]

        def _lane_total(v):
            r = v
            for sh in (8, 4, 2, 1):
                idx = (lanes + sh) % L
                r = r + r.at[idx].get(mode="promise_in_bounds",
                                      unique_indices=True)
            return r

        def emit_edge_loop(glb, grb, cbb, nedges):
            av = [attv[pl.ds(h * 16, 16)] for h in range(8)]
            ohv = [onehot(h) for h in range(8)]

            @plsc.parallel_loop(0, nedges, unroll=2)
            def _edges(e):
                glv = [glb[e, pl.ds(h * 16, 16)] for h in range(8)]
                grv = [grb[e, pl.ds(h * 16, 16)] for h in range(8)]
                nidv = didxp[pl.ds(e, L)]
                x = [glv[h] + grv[h] for h in range(8)]
                m = [jnp.maximum(x[h], 0.2 * x[h]) for h in range(8)]
                if heads == 8:
                    r = [m[h] * av[h] for h in range(8)]
                    for sh in (8, 4, 2, 1):
                        idx = (lanes + sh) % L
                        r = [rh + rh.at[idx].get(mode="promise_in_bounds",
                                                 unique_indices=True)
                             for rh in r]
                    ev = [jnp.exp(rh) for rh in r]
                    out = [glv[h] * ev[h] for h in range(8)]
                    exrow = ev[0] * ohv[0]
                    for h in range(1, 8):
                        exrow = exrow + ev[h] * ohv[h]
                else:
                    ps = m[0] * av[0]
                    for h in range(1, 8):
                        ps = ps + m[h] * av[h]
                    ev = jnp.exp(_lane_total(ps))
                    out = [glv[h] * ev for h in range(8)]
                    exrow = ev * ohv[0]
                for h in range(8):
                    cbb[e, pl.ds(h * 16, 16)] = out[h]
                vslot = jnp.bitwise_and(nidv[0], 7)
                for i in range(8):
                    grb[e, pl.ds(i * 16, 16)] = zero16
                grb[e, pl.ds(vslot * 16, 16)] = exrow

        def start(g, sidxb, didxb, glb, grb, sem_l, sem_r, n):
            base = pl.multiple_of(ebase + g * B, 8)
            if n == B:
                pltpu.sync_copy(src_hbm.at[pl.ds(base, B)], sidxb)
                pltpu.sync_copy(dst_hbm.at[pl.ds(base, B)], didxb)
            else:
                pltpu.sync_copy(src_hbm.at[pl.ds(base, n)],
                                sidxb.at[pl.ds(0, n)])
                pltpu.sync_copy(dst_hbm.at[pl.ds(base, n)],
                                didxb.at[pl.ds(0, n)])
            pltpu.async_copy(xl_hbm.at[sidxb], glb, sem_l)
            pltpu.async_copy(xr_hbm.at[didxb], grb, sem_r)

        def wait(sidxb, didxb, glb, grb, sem_l, sem_r):
            pltpu.make_async_copy(xl_hbm.at[sidxb], glb, sem_l).wait()
            pltpu.make_async_copy(xr_hbm.at[didxb], grb, sem_r).wait()

        def compute(didxb, didx2b, glb, grb, cbb, semS, nedges):
            for kk in range(B // L):
                dv = didxb[pl.ds(kk * L, L)]
                didxp[pl.ds(kk * L, L)] = dv
                didx2b[pl.ds(kk * L, L)] = NP + lax.shift_right_logical(dv, 3)
            emit_edge_loop(glb, grb, cbb, nedges)
            if nedges < B:
                def zb(e, carry3):
                    for i in range(8):
                        cbb[e, pl.ds(i * 16, 16)] = zero16
                        grb[e, pl.ds(i * 16, 16)] = zero16
                    return carry3
                lax.fori_loop(nedges, B, zb, 0, unroll=4)
            pltpu.async_copy(cbb, acc.at[didxb], sem=semS, add=True)
            pltpu.async_copy(grb, acc.at[didx2b], sem=semS, add=True)

        def drain(didxb, didx2b, glb, grb, cbb, semS):
            pltpu.make_async_copy(cbb, acc.at[didxb], semS).wait()
            pltpu.make_async_copy(grb, acc.at[didx2b], semS).wait()

        start(0, sidxA, didxA, glA, grA, semA1, semA2, B)

        def pair_body(gg, carry):
            g0 = gg * 2

            @pl.when(gg > 0)
            def _():
                drain(didxB, didx2B, glB, grB, cbB, semSB)

            start(g0 + 1, sidxB, didxB, glB, grB, semB1, semB2, B)
            wait(sidxA, didxA, glA, grA, semA1, semA2)
            compute(didxA, didx2A, glA, grA, cbA, semSA, B)

            @pl.when(gg < NPAIR - 1)
            def _():
                drain(didxA, didx2A, glA, grA, cbA, semSA)
                start(g0 + 2, sidxA, didxA, glA, grA, semA1, semA2, B)

            wait(sidxB, didxB, glB, grB, semB1, semB2)
            compute(didxB, didx2B, glB, grB, cbB, semSB, B)
            return carry

        lax.fori_loop(0, NPAIR, pair_body, 0)
        drain(didxA, didx2A, glA, grA, cbA, semSA)
        start(NCH_MAIN, sidxA, didxA, glA, grA, semA1, semA2, TAIL)
        wait(sidxA, didxA, glA, grA, semA1, semA2)
        compute(didxA, didx2A, glA, grA, cbA, semSA, TAIL)
        drain(didxA, didx2A, glA, grA, cbA, semSA)
        drain(didxB, didx2B, glB, grB, cbB, semSB)
        plsc.subcore_barrier()
        pltpu.sync_copy(acc.at[pl.ds(s * RPS, RPS)],
                        out_hbm.at[c, pl.ds(s * RPS, RPS)])

    return k


_edge8 = _make_edge_kernel(8)
_edge1 = _make_edge_kernel(1)


def _pre_body(x_ref, wl_ref, wr_ref, xl_ref, xr_ref):
    xv = x_ref[...]
    xl_ref[...] = jnp.dot(xv, wl_ref[...], preferred_element_type=jnp.float32)
    xr_ref[...] = jnp.dot(xv, wr_ref[...], preferred_element_type=jnp.float32)


def _pre(x, wl, wr):
    n = x.shape[0]
    return pl.pallas_call(
        _pre_body,
        out_shape=(jax.ShapeDtypeStruct((n, D), jnp.float32),
                   jax.ShapeDtypeStruct((n, D), jnp.float32)),
    )(x, wl, wr)


def _mid_body(p_ref, den_ref, b1_ref, wl_ref, wr_ref, sel_ref,
              xl2_ref, xr2_ref):
    contrib = (p_ref[0] + p_ref[1])[:N]
    dsum = den_ref[0] + den_ref[1]
    r = 1.0 / (dsum + 1e-16)
    rr = jnp.dot(r, sel_ref[...], preferred_element_type=jnp.float32)
    hv = jnp.maximum(contrib * rr + b1_ref[...], 0.0)
    xl2_ref[...] = jnp.dot(hv, wl_ref[...], preferred_element_type=jnp.float32)
    xr2_ref[...] = jnp.dot(hv, wr_ref[...], preferred_element_type=jnp.float32)


def _mid(p, den, b1, wl2, wr2, sel):
    return pl.pallas_call(
        _mid_body,
        out_shape=(jax.ShapeDtypeStruct((N, D), jnp.float32),
                   jax.ShapeDtypeStruct((N, D), jnp.float32)),
    )(p, den, b1, wl2, wr2, sel)


def _fin_body(p_ref, den_ref, b2_ref, wc1_ref, bc1_ref, wc2_ref, bc2_ref,
              emb_ref, log_ref):
    contrib = (p_ref[0] + p_ref[1])[:N]
    dsum = den_ref[0] + den_ref[1]
    emb = contrib / (dsum[:, 0:1] + 1e-16) + b2_ref[...]
    emb_ref[...] = emb
    t = jnp.maximum(
        jnp.dot(emb, wc1_ref[...], preferred_element_type=jnp.float32)
        + bc1_ref[...], 0.0)
    log_ref[...] = (jnp.dot(t, wc2_ref[...], preferred_element_type=jnp.float32)
                    + bc2_ref[...])


def _fin(p, den, b2, wc1, bc1, wc2, bc2):
    return pl.pallas_call(
        _fin_body,
        out_shape=(jax.ShapeDtypeStruct((N, D), jnp.float32),
                   jax.ShapeDtypeStruct((N, 2), jnp.float32)),
    )(p, den, b2, wc1, bc1, wc2, bc2)


def _split_den(p):
    den = (p[:, NP:NP + N // 8].reshape(NC, N // 8, 8, 16)[:, :, :, :8]
           .reshape(NC, N, 8))
    return p, den


def kernel(x, edge_index, edge_attr, Wl1, Wr1, att1, b1, Wl2, Wr2, att2, b2,
           We1, be1, We2, be2, Wc1, bc1, Wc2, bc2):
    src = edge_index[0]
    dst = edge_index[1]
    z1 = jnp.zeros((NPD, D), jnp.float32)
    sel = jnp.kron(jnp.eye(8, dtype=jnp.float32),
                   jnp.ones((1, 16), jnp.float32))

    onehot_flat = jnp.eye(8, 16, dtype=jnp.float32).reshape(-1)
    aux1 = jnp.concatenate([att1.reshape(-1), onehot_flat])
    aux2 = jnp.concatenate([att2.reshape(-1), onehot_flat])

    xl1, xr1 = _pre(x, Wl1, Wr1)
    p1 = _edge8(xl1, xr1, src, dst, aux1, z1)
    p1, den1 = _split_den(p1)
    xl2, xr2 = _mid(p1, den1, b1, Wl2, Wr2, sel)
    p2 = _edge1(xl2, xr2, src, dst, aux2, z1)
    p2, den2 = _split_den(p2)
    emb, logits = _fin(p2, den2, b2, Wc1, bc1, Wc2, bc2)
    return emb, logits


# speedup vs baseline: 1.0706x; 1.0706x over previous
import functools

import jax
import jax.numpy as jnp
from jax import lax
from jax.experimental import pallas as pl
from jax.experimental.pallas import tpu as pltpu
from jax.experimental.pallas import tpu_sc as plsc

N = 10000
E = 320000
D = 128
NP = 10000
DROWS = 1264
NPD = NP + DROWS
NC, NS, L = 2, 16, 16
NW = NC * NS
EPT = E // NW
B = 48
NCH_MAIN = EPT // B
TAIL = EPT - NCH_MAIN * B
NPAIR = NCH_MAIN // 2
RPS = NPD // NS


def _make_edge_kernel(heads):
    mesh = plsc.VectorSubcoreMesh(core_axis_name="c", subcore_axis_name="s",
                                  num_cores=NC, num_subcores=NS)

    @functools.partial(
        pl.kernel,
        mesh=mesh,
        out_type=jax.ShapeDtypeStruct((NC, NPD, D), jnp.float32),
        scratch_types=[
            pltpu.VMEM((B,), jnp.int32),
            pltpu.VMEM((B,), jnp.int32),
            pltpu.VMEM((B,), jnp.int32),
            pltpu.VMEM((B,), jnp.int32),
            pltpu.VMEM((B + L,), jnp.int32),
            pltpu.VMEM((B,), jnp.int32),
            pltpu.VMEM((B,), jnp.int32),
            pltpu.VMEM((B, D), jnp.float32),
            pltpu.VMEM((B, D), jnp.float32),
            pltpu.VMEM((B, D), jnp.float32),
            pltpu.VMEM((B, D), jnp.float32),
            pltpu.VMEM((B, D), jnp.float32),
            pltpu.VMEM((B, D), jnp.float32),
            pltpu.VMEM((2 * D,), jnp.float32),
            pltpu.VMEM_SHARED((NPD, D), jnp.float32),
            pltpu.SemaphoreType.DMA,
            pltpu.SemaphoreType.DMA,
            pltpu.SemaphoreType.DMA,
            pltpu.SemaphoreType.DMA,
            pltpu.SemaphoreType.DMA,
            pltpu.SemaphoreType.DMA,
        ],
    )
    def k(xl_hbm, xr_hbm, src_hbm, dst_hbm, att_hbm, z1_hbm, out_hbm,
          sidxA, sidxB, didxA, didxB, didxp, didx2A, didx2B,
          glA, glB, grA, grB, cbA, cbB, attv, acc,
          semA1, semA2, semB1, semB2, semSA, semSB):
        c = lax.axis_index("c")
        s = lax.axis_index("s")
        wid = s * NC + c
        pltpu.sync_copy(z1_hbm.at[pl.ds(s * RPS, RPS)],
                        acc.at[pl.ds(s * RPS, RPS)])
        pltpu.sync_copy(att_hbm, attv)
        plsc.subcore_barrier()

        ebase = wid * EPT
        lanes = lax.iota(jnp.int32, L)
        zero16 = jnp.zeros((L,), jnp.float32)

        def onehot(h):
            return attv[pl.ds(D + h * 16, 16)]

        def _lane_total(v):
            r = v
            for sh in (8, 4, 2, 1):
                idx = (lanes + sh) % L
                r = r + r.at[idx].get(mode="promise_in_bounds",
                                      unique_indices=True)
            return r

        def emit_edge_loop(glb, grb, cbb, nedges):
            @plsc.parallel_loop(0, nedges, unroll=2)
            def _edges(e):
                glv = [glb[e, pl.ds(h * 16, 16)] for h in range(8)]
                grv = [grb[e, pl.ds(h * 16, 16)] for h in range(8)]
                av = [attv[pl.ds(h * 16, 16)] for h in range(8)]
                nidv = didxp[pl.ds(e, L)]
                x = [glv[h] + grv[h] for h in range(8)]
                m = [jnp.maximum(x[h], 0.2 * x[h]) for h in range(8)]
                if heads == 8:
                    r = [m[h] * av[h] for h in range(8)]
                    for sh in (8, 4, 2, 1):
                        idx = (lanes + sh) % L
                        r = [rh + rh.at[idx].get(mode="promise_in_bounds",
                                                 unique_indices=True)
                             for rh in r]
                    ev = [jnp.exp(rh) for rh in r]
                    out = [glv[h] * ev[h] for h in range(8)]
                    exrow = ev[0] * onehot(0)
                    for h in range(1, 8):
                        exrow = exrow + ev[h] * onehot(h)
                else:
                    ps = m[0] * av[0]
                    for h in range(1, 8):
                        ps = ps + m[h] * av[h]
                    ev = jnp.exp(_lane_total(ps))
                    out = [glv[h] * ev for h in range(8)]
                    exrow = ev * onehot(0)
                for h in range(8):
                    cbb[e, pl.ds(h * 16, 16)] = out[h]
                vslot = jnp.bitwise_and(nidv[0], 7)
                for i in range(8):
                    grb[e, pl.ds(i * 16, 16)] = zero16
                grb[e, pl.ds(vslot * 16, 16)] = exrow

        def start(g, sidxb, didxb, glb, grb, sem_l, sem_r, n):
            base = pl.multiple_of(ebase + g * B, 8)
            if n == B:
                pltpu.sync_copy(src_hbm.at[pl.ds(base, B)], sidxb)
                pltpu.sync_copy(dst_hbm.at[pl.ds(base, B)], didxb)
            else:
                pltpu.sync_copy(src_hbm.at[pl.ds(base, n)],
                                sidxb.at[pl.ds(0, n)])
                pltpu.sync_copy(dst_hbm.at[pl.ds(base, n)],
                                didxb.at[pl.ds(0, n)])
            pltpu.async_copy(xl_hbm.at[sidxb], glb, sem_l)
            pltpu.async_copy(xr_hbm.at[didxb], grb, sem_r)

        def wait(sidxb, didxb, glb, grb, sem_l, sem_r):
            pltpu.make_async_copy(xl_hbm.at[sidxb], glb, sem_l).wait()
            pltpu.make_async_copy(xr_hbm.at[didxb], grb, sem_r).wait()

        def compute(didxb, didx2b, glb, grb, cbb, semS, nedges):
            for kk in range(B // L):
                dv = didxb[pl.ds(kk * L, L)]
                didxp[pl.ds(kk * L, L)] = dv
                didx2b[pl.ds(kk * L, L)] = NP + lax.shift_right_logical(dv, 3)
            emit_edge_loop(glb, grb, cbb, nedges)
            if nedges < B:
                def zb(e, carry3):
                    for i in range(8):
                        cbb[e, pl.ds(i * 16, 16)] = zero16
                        grb[e, pl.ds(i * 16, 16)] = zero16
                    return carry3
                lax.fori_loop(nedges, B, zb, 0, unroll=4)
            pltpu.async_copy(cbb, acc.at[didxb], sem=semS, add=True)
            pltpu.async_copy(grb, acc.at[didx2b], sem=semS, add=True)

        def drain(didxb, didx2b, glb, grb, cbb, semS):
            pltpu.make_async_copy(cbb, acc.at[didxb], semS).wait()
            pltpu.make_async_copy(grb, acc.at[didx2b], semS).wait()

        start(0, sidxA, didxA, glA, grA, semA1, semA2, B)

        def pair_body(gg, carry):
            g0 = gg * 2

            @pl.when(gg > 0)
            def _():
                drain(didxB, didx2B, glB, grB, cbB, semSB)

            start(g0 + 1, sidxB, didxB, glB, grB, semB1, semB2, B)
            wait(sidxA, didxA, glA, grA, semA1, semA2)
            compute(didxA, didx2A, glA, grA, cbA, semSA, B)

            @pl.when(gg < NPAIR - 1)
            def _():
                drain(didxA, didx2A, glA, grA, cbA, semSA)
                start(g0 + 2, sidxA, didxA, glA, grA, semA1, semA2, B)

            wait(sidxB, didxB, glB, grB, semB1, semB2)
            compute(didxB, didx2B, glB, grB, cbB, semSB, B)
            return carry

        lax.fori_loop(0, NPAIR, pair_body, 0)
        drain(didxA, didx2A, glA, grA, cbA, semSA)
        start(NCH_MAIN, sidxA, didxA, glA, grA, semA1, semA2, TAIL)
        wait(sidxA, didxA, glA, grA, semA1, semA2)
        compute(didxA, didx2A, glA, grA, cbA, semSA, TAIL)
        drain(didxA, didx2A, glA, grA, cbA, semSA)
        drain(didxB, didx2B, glB, grB, cbB, semSB)
        plsc.subcore_barrier()
        pltpu.sync_copy(acc.at[pl.ds(s * RPS, RPS)],
                        out_hbm.at[c, pl.ds(s * RPS, RPS)])

    return k


_edge8 = _make_edge_kernel(8)
_edge1 = _make_edge_kernel(1)


def _pre_body(x_ref, wl_ref, wr_ref, xl_ref, xr_ref):
    xv = x_ref[...]
    xl_ref[...] = jnp.dot(xv, wl_ref[...], preferred_element_type=jnp.float32)
    xr_ref[...] = jnp.dot(xv, wr_ref[...], preferred_element_type=jnp.float32)


def _pre(x, wl, wr):
    n = x.shape[0]
    return pl.pallas_call(
        _pre_body,
        out_shape=(jax.ShapeDtypeStruct((n, D), jnp.float32),
                   jax.ShapeDtypeStruct((n, D), jnp.float32)),
    )(x, wl, wr)


def _mid_body(p_ref, den_ref, b1_ref, wl_ref, wr_ref, sel_ref,
              xl2_ref, xr2_ref):
    contrib = (p_ref[0] + p_ref[1])[:N]
    dsum = den_ref[0] + den_ref[1]
    r = 1.0 / (dsum + 1e-16)
    rr = jnp.dot(r, sel_ref[...], preferred_element_type=jnp.float32)
    hv = jnp.maximum(contrib * rr + b1_ref[...], 0.0)
    xl2_ref[...] = jnp.dot(hv, wl_ref[...], preferred_element_type=jnp.float32)
    xr2_ref[...] = jnp.dot(hv, wr_ref[...], preferred_element_type=jnp.float32)


def _mid(p, den, b1, wl2, wr2, sel):
    return pl.pallas_call(
        _mid_body,
        out_shape=(jax.ShapeDtypeStruct((N, D), jnp.float32),
                   jax.ShapeDtypeStruct((N, D), jnp.float32)),
    )(p, den, b1, wl2, wr2, sel)


def _fin_body(p_ref, den_ref, b2_ref, wc1_ref, bc1_ref, wc2_ref, bc2_ref,
              emb_ref, log_ref):
    contrib = (p_ref[0] + p_ref[1])[:N]
    dsum = den_ref[0] + den_ref[1]
    emb = contrib / (dsum[:, 0:1] + 1e-16) + b2_ref[...]
    emb_ref[...] = emb
    t = jnp.maximum(
        jnp.dot(emb, wc1_ref[...], preferred_element_type=jnp.float32)
        + bc1_ref[...], 0.0)
    log_ref[...] = (jnp.dot(t, wc2_ref[...], preferred_element_type=jnp.float32)
                    + bc2_ref[...])


def _fin(p, den, b2, wc1, bc1, wc2, bc2):
    return pl.pallas_call(
        _fin_body,
        out_shape=(jax.ShapeDtypeStruct((N, D), jnp.float32),
                   jax.ShapeDtypeStruct((N, 2), jnp.float32)),
    )(p, den, b2, wc1, bc1, wc2, bc2)


def _split_den(p):
    den = (p[:, NP:NP + N // 8].reshape(NC, N // 8, 8, 16)[:, :, :, :8]
           .reshape(NC, N, 8))
    return p, den


def kernel(x, edge_index, edge_attr, Wl1, Wr1, att1, b1, Wl2, Wr2, att2, b2,
           We1, be1, We2, be2, Wc1, bc1, Wc2, bc2):
    src = edge_index[0]
    dst = edge_index[1]
    z1 = jnp.zeros((NPD, D), jnp.float32)
    sel = jnp.kron(jnp.eye(8, dtype=jnp.float32),
                   jnp.ones((1, 16), jnp.float32))

    onehot_flat = jnp.eye(8, 16, dtype=jnp.float32).reshape(-1)
    aux1 = jnp.concatenate([att1.reshape(-1), onehot_flat])
    aux2 = jnp.concatenate([att2.reshape(-1), onehot_flat])

    xl1, xr1 = _pre(x, Wl1, Wr1)
    p1 = _edge8(xl1, xr1, src, dst, aux1, z1)
    p1, den1 = _split_den(p1)
    xl2, xr2 = _mid(p1, den1, b1, Wl2, Wr2, sel)
    p2 = _edge1(xl2, xr2, src, dst, aux2, z1)
    p2, den2 = _split_den(p2)
    emb, logits = _fin(p2, den2, b2, Wc1, bc1, Wc2, bc2)
    return emb, logits
